# baseline (speedup 1.0000x reference)
import jax
import jax.numpy as jnp
from jax import lax
from jax.experimental import pallas as pl
from jax.experimental.pallas import tpu as pltpu

_NCLS = 19
_S = _NCLS + 1
_C = 64
_H = 128
_W = 128
_P = _H * _W
_RB = 32
_NRB = _H // _RB


def _pool_body(feat_ref, seg_ref, fcwp_ref, fcbt_ref, vec_ref):
    f32 = jnp.float32
    x = feat_ref[0, 0]
    seg_r = seg_ref[0, 0]

    iota_sr = lax.broadcasted_iota(jnp.int32, (_S, _P), 0)
    oh = (seg_r == iota_sr - 1).astype(f32)
    sums = lax.dot_general(x, oh, (((1,), (1,)), ((), ())),
                           preferred_element_type=f32)
    ones_row = jnp.ones((1, _P), f32)
    area = lax.dot_general(ones_row, oh, (((1,), (1,)), ((), ())),
                           preferred_element_type=f32)
    mu = jnp.where(area > 0.0, sums / jnp.maximum(area, 1.0), 0.0)
    glob = jnp.sum(sums, axis=1, keepdims=True) * (1.0 / _P)

    fcwp_flat = fcwp_ref[...].reshape(_C * _S, _C)
    prod = lax.dot_general(fcwp_flat, mu, (((1,), (0,)), ((), ())),
                           preferred_element_type=f32)
    prod3 = prod.reshape(_C, _S, _S)
    sel = (lax.broadcasted_iota(jnp.int32, (_C, _S, _S), 1) ==
           lax.broadcasted_iota(jnp.int32, (_C, _S, _S), 2))
    fc_lin = jnp.sum(jnp.where(sel, prod3, 0.0), axis=2) + fcbt_ref[...]
    fc_out = jnp.maximum(fc_lin, 0.0)

    fc0 = jnp.maximum(
        lax.dot_general(fcwp_ref[:, 0, :], glob, (((1,), (0,)), ((), ())),
                        preferred_element_type=f32) + fcbt_ref[:, 0:1], 0.0)

    present = jnp.sum(mu, axis=0, keepdims=True)
    sidx = lax.broadcasted_iota(jnp.int32, (1, _S), 1)
    use_fc = jnp.logical_and(sidx != 0, present != 0.0)
    vec_ref[0, 0] = jnp.where(use_fc, fc_out, fc0)


def _conv_body(seg_ref, vec_ref, wt_ref, bias_ref, out_ref, fpad_ref):
    f32 = jnp.float32
    pid0 = pl.program_id(0)
    pid1 = pl.program_id(1)
    rb = pl.program_id(2)

    @pl.when(jnp.logical_and(jnp.logical_and(pid0 == 0, pid1 == 0), rb == 0))
    def _zero():
        fpad_ref[...] = jnp.zeros((_H + 2, _W + 2, _C), jnp.bfloat16)

    @pl.when(rb == 0)
    def _build_f():
        seg_c = seg_ref[0, 0]
        iota_sr = lax.broadcasted_iota(jnp.int32, (_S, _P), 0)
        oh_t = (seg_c == iota_sr - 1).astype(f32)
        oh = jnp.transpose(oh_t)
        f = lax.dot_general(oh, vec_ref[0, 0], (((1,), (1,)), ((), ())),
                            preferred_element_type=f32)
        fpad_ref[1:_H + 1, 1:_W + 1, :] = f.astype(jnp.bfloat16).reshape(_H, _W, _C)

    acc = None
    for t in range(9):
        ky, kx = divmod(t, 3)
        xs = fpad_ref[pl.ds(rb * _RB + ky, _RB), kx:kx + _W, :]
        xs = xs.reshape(_RB * _W, _C)
        m = lax.dot_general(xs, wt_ref[t], (((1,), (0,)), ((), ())),
                            preferred_element_type=f32)
        acc = m if acc is None else acc + m
    out_ref[0, 0] = jnp.maximum(acc + bias_ref[...], 0.0)


def _src_idx(p):
    return p % 2


def _tgt_idx(p):
    return (p % 2 + p // 2) % 2


def kernel(feature_A, feature_B, segmap_A, segmap_B, fc_w, fc_b,
           gamma_w, gamma_b, beta_w, beta_b):
    feats = jnp.stack([feature_A, feature_B])
    b = feats.shape[1]
    feats = feats.reshape(2, b, _C, _P)
    segs = jnp.stack([segmap_A, segmap_B]).astype(jnp.int32)
    seg_row = segs.reshape(2, b, 1, _P)
    fcwp = jnp.transpose(fc_w, (1, 0, 2))
    fcbt = jnp.transpose(fc_b)

    vecs = pl.pallas_call(
        _pool_body,
        grid=(2, b),
        in_specs=[
            pl.BlockSpec((1, 1, _C, _P), lambda d, i: (d, i, 0, 0)),
            pl.BlockSpec((1, 1, 1, _P), lambda d, i: (d, i, 0, 0)),
            pl.BlockSpec((_C, _S, _C), lambda d, i: (0, 0, 0)),
            pl.BlockSpec((_C, _S), lambda d, i: (0, 0)),
        ],
        out_specs=pl.BlockSpec((1, 1, _C, _S), lambda d, i: (d, i, 0, 0)),
        out_shape=jax.ShapeDtypeStruct((2, b, _C, _S), jnp.float32),
    )(feats, seg_row, fcwp, fcbt)

    w2 = jnp.concatenate([gamma_w, beta_w], axis=0)
    wt = jnp.transpose(w2, (2, 3, 1, 0)).reshape(9, _C, 2 * _C).astype(jnp.bfloat16)
    bias2 = jnp.concatenate([gamma_b, beta_b])[None, :]

    out = pl.pallas_call(
        _conv_body,
        grid=(4, b, _NRB),
        in_specs=[
            pl.BlockSpec((1, 1, 1, _P), lambda p, i, r: (_src_idx(p), i, 0, 0)),
            pl.BlockSpec((1, 1, _C, _S), lambda p, i, r: (_tgt_idx(p), i, 0, 0)),
            pl.BlockSpec((9, _C, 2 * _C), lambda p, i, r: (0, 0, 0)),
            pl.BlockSpec((1, 2 * _C), lambda p, i, r: (0, 0)),
        ],
        out_specs=pl.BlockSpec((1, 1, _RB * _W, 2 * _C),
                               lambda p, i, r: (p, i, r, 0)),
        out_shape=jax.ShapeDtypeStruct((4, b, _P, 2 * _C), jnp.float32),
        scratch_shapes=[pltpu.VMEM((_H + 2, _W + 2, _C), jnp.bfloat16)],
    )(seg_row, vecs, wt, bias2)

    gam = out[..., :_C].transpose(0, 1, 3, 2).reshape(4, b, _C, _H, _W)
    bet = out[..., _C:].transpose(0, 1, 3, 2).reshape(4, b, _C, _H, _W)
    return jnp.concatenate([gam, bet], axis=0)


# speedup vs baseline: 1.1197x; 1.1197x over previous
import jax
import jax.numpy as jnp
from jax import lax
from jax.experimental import pallas as pl
from jax.experimental.pallas import tpu as pltpu

_NCLS = 19
_S = _NCLS + 1
_C = 64
_H = 128
_W = 128
_P = _H * _W
_RB = 32
_NRB = _H // _RB


def _pool_body(feat_ref, seg_ref, fcwp_ref, fcbt_ref, vec_ref):
    f32 = jnp.float32
    x = feat_ref[0, 0]
    seg_r = seg_ref[0, 0]

    iota_sr = lax.broadcasted_iota(jnp.int32, (_S, _P), 0)
    oh = (seg_r == iota_sr - 1).astype(f32)
    sums = lax.dot_general(x, oh, (((1,), (1,)), ((), ())),
                           preferred_element_type=f32)
    ones_row = jnp.ones((1, _P), f32)
    area = lax.dot_general(ones_row, oh, (((1,), (1,)), ((), ())),
                           preferred_element_type=f32)
    mu = jnp.where(area > 0.0, sums / jnp.maximum(area, 1.0), 0.0)
    glob = jnp.sum(sums, axis=1, keepdims=True) * (1.0 / _P)

    fcwp_flat = fcwp_ref[...].reshape(_C * _S, _C)
    prod = lax.dot_general(fcwp_flat, mu, (((1,), (0,)), ((), ())),
                           preferred_element_type=f32)
    prod3 = prod.reshape(_C, _S, _S)
    sel = (lax.broadcasted_iota(jnp.int32, (_C, _S, _S), 1) ==
           lax.broadcasted_iota(jnp.int32, (_C, _S, _S), 2))
    fc_lin = jnp.sum(jnp.where(sel, prod3, 0.0), axis=2) + fcbt_ref[...]
    fc_out = jnp.maximum(fc_lin, 0.0)

    fc0 = jnp.maximum(
        lax.dot_general(fcwp_ref[:, 0, :], glob, (((1,), (0,)), ((), ())),
                        preferred_element_type=f32) + fcbt_ref[:, 0:1], 0.0)

    present = jnp.sum(mu, axis=0, keepdims=True)
    sidx = lax.broadcasted_iota(jnp.int32, (1, _S), 1)
    use_fc = jnp.logical_and(sidx != 0, present != 0.0)
    vec_ref[0, 0] = jnp.where(use_fc, fc_out, fc0)


def _conv_body(seg_ref, vec_ref, wt_ref, bias_ref, out_ref, fpad_ref):
    f32 = jnp.float32
    pid0 = pl.program_id(0)
    pid1 = pl.program_id(1)
    rb = pl.program_id(2)

    @pl.when(jnp.logical_and(jnp.logical_and(pid0 == 0, pid1 == 0), rb == 0))
    def _zero():
        fpad_ref[...] = jnp.zeros((_H + 2, _W + 2, _C), f32)

    @pl.when(rb == 0)
    def _build_f():
        seg_c = seg_ref[0, 0]
        iota_sr = lax.broadcasted_iota(jnp.int32, (_S, _P), 0)
        oh_t = (seg_c == iota_sr - 1).astype(f32)
        oh = jnp.transpose(oh_t)
        f = lax.dot_general(oh, vec_ref[0, 0], (((1,), (1,)), ((), ())),
                            preferred_element_type=f32)
        fpad_ref[1:_H + 1, 1:_W + 1, :] = f.reshape(_H, _W, _C)

    acc = None
    for t in range(9):
        ky, kx = divmod(t, 3)
        xs = fpad_ref[pl.ds(rb * _RB + ky, _RB), kx:kx + _W, :]
        xs = xs.reshape(_RB * _W, _C)
        m = lax.dot_general(xs, wt_ref[t], (((1,), (0,)), ((), ())),
                            preferred_element_type=f32)
        acc = m if acc is None else acc + m
    out_ref[0, 0] = jnp.maximum(acc + bias_ref[...], 0.0)


def _src_idx(p):
    return p % 2


def _tgt_idx(p):
    return (p % 2 + p // 2) % 2


def kernel(feature_A, feature_B, segmap_A, segmap_B, fc_w, fc_b,
           gamma_w, gamma_b, beta_w, beta_b):
    feats = jnp.stack([feature_A, feature_B])
    b = feats.shape[1]
    feats = feats.reshape(2, b, _C, _P)
    segs = jnp.stack([segmap_A, segmap_B]).astype(jnp.int32)
    seg_row = segs.reshape(2, b, 1, _P)
    fcwp = jnp.transpose(fc_w, (1, 0, 2))
    fcbt = jnp.transpose(fc_b)

    vecs = pl.pallas_call(
        _pool_body,
        grid=(2, b),
        in_specs=[
            pl.BlockSpec((1, 1, _C, _P), lambda d, i: (d, i, 0, 0)),
            pl.BlockSpec((1, 1, 1, _P), lambda d, i: (d, i, 0, 0)),
            pl.BlockSpec((_C, _S, _C), lambda d, i: (0, 0, 0)),
            pl.BlockSpec((_C, _S), lambda d, i: (0, 0)),
        ],
        out_specs=pl.BlockSpec((1, 1, _C, _S), lambda d, i: (d, i, 0, 0)),
        out_shape=jax.ShapeDtypeStruct((2, b, _C, _S), jnp.float32),
    )(feats, seg_row, fcwp, fcbt)

    w2 = jnp.concatenate([gamma_w, beta_w], axis=0)
    wt = jnp.transpose(w2, (2, 3, 1, 0)).reshape(9, _C, 2 * _C)
    bias2 = jnp.concatenate([gamma_b, beta_b])[None, :]

    out = pl.pallas_call(
        _conv_body,
        grid=(4, b, _NRB),
        in_specs=[
            pl.BlockSpec((1, 1, 1, _P), lambda p, i, r: (_src_idx(p), i, 0, 0)),
            pl.BlockSpec((1, 1, _C, _S), lambda p, i, r: (_tgt_idx(p), i, 0, 0)),
            pl.BlockSpec((9, _C, 2 * _C), lambda p, i, r: (0, 0, 0)),
            pl.BlockSpec((1, 2 * _C), lambda p, i, r: (0, 0)),
        ],
        out_specs=pl.BlockSpec((1, 1, _RB * _W, 2 * _C),
                               lambda p, i, r: (p, i, r, 0)),
        out_shape=jax.ShapeDtypeStruct((4, b, _P, 2 * _C), jnp.float32),
        scratch_shapes=[pltpu.VMEM((_H + 2, _W + 2, _C), jnp.float32)],
    )(seg_row, vecs, wt, bias2)

    gam = out[..., :_C].transpose(0, 1, 3, 2).reshape(4, b, _C, _H, _W)
    bet = out[..., _C:].transpose(0, 1, 3, 2).reshape(4, b, _C, _H, _W)
    return jnp.concatenate([gam, bet], axis=0)


# speedup vs baseline: 1.8093x; 1.6158x over previous
import jax
import jax.numpy as jnp
from jax import lax
from jax.experimental import pallas as pl
from jax.experimental.pallas import tpu as pltpu

_NCLS = 19
_S = _NCLS + 1
_C = 64
_H = 128
_W = 128
_P = _H * _W
_RB = 32
_NRB = _H // _RB


def _pool_body(feat_ref, seg_ref, fcwp_ref, fcbt_ref, vec_ref):
    f32 = jnp.float32
    x = feat_ref[0, 0]
    seg_r = seg_ref[0, 0]

    iota_sr = lax.broadcasted_iota(jnp.int32, (_S, _P), 0)
    oh = (seg_r == iota_sr - 1).astype(f32)
    sums = lax.dot_general(x, oh, (((1,), (1,)), ((), ())),
                           preferred_element_type=f32)
    ones_row = jnp.ones((1, _P), f32)
    area = lax.dot_general(ones_row, oh, (((1,), (1,)), ((), ())),
                           preferred_element_type=f32)
    mu = jnp.where(area > 0.0, sums / jnp.maximum(area, 1.0), 0.0)
    glob = jnp.sum(sums, axis=1, keepdims=True) * (1.0 / _P)

    fcwp_flat = fcwp_ref[...].reshape(_C * _S, _C)
    prod = lax.dot_general(fcwp_flat, mu, (((1,), (0,)), ((), ())),
                           preferred_element_type=f32)
    prod3 = prod.reshape(_C, _S, _S)
    sel = (lax.broadcasted_iota(jnp.int32, (_C, _S, _S), 1) ==
           lax.broadcasted_iota(jnp.int32, (_C, _S, _S), 2))
    fc_lin = jnp.sum(jnp.where(sel, prod3, 0.0), axis=2) + fcbt_ref[...]
    fc_out = jnp.maximum(fc_lin, 0.0)

    fc0 = jnp.maximum(
        lax.dot_general(fcwp_ref[:, 0, :], glob, (((1,), (0,)), ((), ())),
                        preferred_element_type=f32) + fcbt_ref[:, 0:1], 0.0)

    present = jnp.sum(mu, axis=0, keepdims=True)
    sidx = lax.broadcasted_iota(jnp.int32, (1, _S), 1)
    use_fc = jnp.logical_and(sidx != 0, present != 0.0)
    vec_ref[0, 0] = jnp.where(use_fc, fc_out, fc0)


def _conv_body(seg_ref, vec_ref, wt_ref, bias_ref, out_ref, fpad_ref):
    f32 = jnp.float32
    pid0 = pl.program_id(0)
    pid1 = pl.program_id(1)
    rb = pl.program_id(2)

    @pl.when(jnp.logical_and(jnp.logical_and(pid0 == 0, pid1 == 0), rb == 0))
    def _zero():
        fpad_ref[...] = jnp.zeros((_H + 2, _W + 2, _C), f32)

    @pl.when(rb == 0)
    def _build_f():
        seg_c = seg_ref[0, 0]
        iota_sr = lax.broadcasted_iota(jnp.int32, (_S, _P), 0)
        oh_t = (seg_c == iota_sr - 1).astype(f32)
        oh = jnp.transpose(oh_t)
        f = lax.dot_general(oh, vec_ref[0, 0], (((1,), (1,)), ((), ())),
                            preferred_element_type=f32)
        fpad_ref[1:_H + 1, 1:_W + 1, :] = f.reshape(_H, _W, _C)

    acc = None
    for t in range(9):
        ky, kx = divmod(t, 3)
        xs = fpad_ref[pl.ds(rb * _RB + ky, _RB), kx:kx + _W, :]
        xs = xs.reshape(_RB * _W, _C)
        m = lax.dot_general(xs, wt_ref[t], (((1,), (0,)), ((), ())),
                            preferred_element_type=f32)
        acc = m if acc is None else acc + m
    res = jnp.transpose(jnp.maximum(acc + bias_ref[...], 0.0))
    out_ref[0, 0, 0] = res[:_C]
    out_ref[1, 0, 0] = res[_C:]


def _src_idx(p):
    return p % 2


def _tgt_idx(p):
    return (p % 2 + p // 2) % 2


def kernel(feature_A, feature_B, segmap_A, segmap_B, fc_w, fc_b,
           gamma_w, gamma_b, beta_w, beta_b):
    feats = jnp.stack([feature_A, feature_B])
    b = feats.shape[1]
    feats = feats.reshape(2, b, _C, _P)
    segs = jnp.stack([segmap_A, segmap_B]).astype(jnp.int32)
    seg_row = segs.reshape(2, b, 1, _P)
    fcwp = jnp.transpose(fc_w, (1, 0, 2))
    fcbt = jnp.transpose(fc_b)

    vecs = pl.pallas_call(
        _pool_body,
        grid=(2, b),
        in_specs=[
            pl.BlockSpec((1, 1, _C, _P), lambda d, i: (d, i, 0, 0)),
            pl.BlockSpec((1, 1, 1, _P), lambda d, i: (d, i, 0, 0)),
            pl.BlockSpec((_C, _S, _C), lambda d, i: (0, 0, 0)),
            pl.BlockSpec((_C, _S), lambda d, i: (0, 0)),
        ],
        out_specs=pl.BlockSpec((1, 1, _C, _S), lambda d, i: (d, i, 0, 0)),
        out_shape=jax.ShapeDtypeStruct((2, b, _C, _S), jnp.float32),
    )(feats, seg_row, fcwp, fcbt)

    w2 = jnp.concatenate([gamma_w, beta_w], axis=0)
    wt = jnp.transpose(w2, (2, 3, 1, 0)).reshape(9, _C, 2 * _C)
    bias2 = jnp.concatenate([gamma_b, beta_b])[None, :]

    out = pl.pallas_call(
        _conv_body,
        grid=(4, b, _NRB),
        in_specs=[
            pl.BlockSpec((1, 1, 1, _P), lambda p, i, r: (_src_idx(p), i, 0, 0)),
            pl.BlockSpec((1, 1, _C, _S), lambda p, i, r: (_tgt_idx(p), i, 0, 0)),
            pl.BlockSpec((9, _C, 2 * _C), lambda p, i, r: (0, 0, 0)),
            pl.BlockSpec((1, 2 * _C), lambda p, i, r: (0, 0)),
        ],
        out_specs=pl.BlockSpec((2, 1, 1, _C, _RB * _W),
                               lambda p, i, r: (0, p, i, 0, r)),
        out_shape=jax.ShapeDtypeStruct((2, 4, b, _C, _P), jnp.float32),
        scratch_shapes=[pltpu.VMEM((_H + 2, _W + 2, _C), jnp.float32)],
    )(seg_row, vecs, wt, bias2)

    return out.reshape(8, b, _C, _H, _W)


# speedup vs baseline: 1.9183x; 1.0603x over previous
import jax
import jax.numpy as jnp
from jax import lax
from jax.experimental import pallas as pl
from jax.experimental.pallas import tpu as pltpu

_NCLS = 19
_S = _NCLS + 1
_C = 64
_H = 128
_W = 128
_P = _H * _W
_RB = 32
_NRB = _H // _RB


def _pool_one(x, seg_r, fcwp_ref, fcbt_ref):
    f32 = jnp.float32
    iota_sr = lax.broadcasted_iota(jnp.int32, (_S, _P), 0)
    oh = (seg_r == iota_sr - 1).astype(f32)
    sums = lax.dot_general(x, oh, (((1,), (1,)), ((), ())),
                           preferred_element_type=f32)
    ones_row = jnp.ones((1, _P), f32)
    area = lax.dot_general(ones_row, oh, (((1,), (1,)), ((), ())),
                           preferred_element_type=f32)
    mu = jnp.where(area > 0.0, sums / jnp.maximum(area, 1.0), 0.0)
    glob = jnp.sum(sums, axis=1, keepdims=True) * (1.0 / _P)

    fcwp_flat = fcwp_ref[...].reshape(_C * _S, _C)
    prod = lax.dot_general(fcwp_flat, mu, (((1,), (0,)), ((), ())),
                           preferred_element_type=f32)
    prod3 = prod.reshape(_C, _S, _S)
    sel = (lax.broadcasted_iota(jnp.int32, (_C, _S, _S), 1) ==
           lax.broadcasted_iota(jnp.int32, (_C, _S, _S), 2))
    fc_lin = jnp.sum(jnp.where(sel, prod3, 0.0), axis=2) + fcbt_ref[...]
    fc_out = jnp.maximum(fc_lin, 0.0)

    fc0 = jnp.maximum(
        lax.dot_general(fcwp_ref[:, 0, :], glob, (((1,), (0,)), ((), ())),
                        preferred_element_type=f32) + fcbt_ref[:, 0:1], 0.0)

    present = jnp.sum(mu, axis=0, keepdims=True)
    sidx = lax.broadcasted_iota(jnp.int32, (1, _S), 1)
    use_fc = jnp.logical_and(sidx != 0, present != 0.0)
    return jnp.where(use_fc, fc_out, fc0)


def _pool_body(featA_ref, featB_ref, segA_ref, segB_ref, fcwp_ref, fcbt_ref,
               vec_ref):
    vec_ref[0, 0] = _pool_one(featA_ref[0], segA_ref[0], fcwp_ref, fcbt_ref)
    vec_ref[1, 0] = _pool_one(featB_ref[0], segB_ref[0], fcwp_ref, fcbt_ref)


def _conv_body(segA_ref, segB_ref, vec_ref, wt_ref, bias_ref, out_ref,
               fpad_ref):
    f32 = jnp.float32
    pid0 = pl.program_id(0)
    pid1 = pl.program_id(1)
    rb = pl.program_id(2)

    @pl.when(jnp.logical_and(jnp.logical_and(pid0 == 0, pid1 == 0), rb == 0))
    def _zero():
        fpad_ref[...] = jnp.zeros((_H + 2, _W + 2, _C), f32)

    @pl.when(rb == 0)
    def _build_f():
        seg_c = jnp.where(pid0 % 2 == 0, segA_ref[0], segB_ref[0])
        iota_sr = lax.broadcasted_iota(jnp.int32, (_S, _P), 0)
        oh_t = (seg_c == iota_sr - 1).astype(f32)
        oh = jnp.transpose(oh_t)
        f = lax.dot_general(oh, vec_ref[0, 0], (((1,), (1,)), ((), ())),
                            preferred_element_type=f32)
        fpad_ref[1:_H + 1, 1:_W + 1, :] = f.reshape(_H, _W, _C)

    acc = None
    for t in range(9):
        ky, kx = divmod(t, 3)
        xs = fpad_ref[pl.ds(rb * _RB + ky, _RB), kx:kx + _W, :]
        xs = xs.reshape(_RB * _W, _C)
        m = lax.dot_general(xs, wt_ref[t], (((1,), (0,)), ((), ())),
                            preferred_element_type=f32)
        acc = m if acc is None else acc + m
    res = jnp.transpose(jnp.maximum(acc + bias_ref[...], 0.0))
    out_ref[0, 0, 0] = res[:_C]
    out_ref[1, 0, 0] = res[_C:]


def _src_idx(p):
    return p % 2


def _tgt_idx(p):
    return (p % 2 + p // 2) % 2


def kernel(feature_A, feature_B, segmap_A, segmap_B, fc_w, fc_b,
           gamma_w, gamma_b, beta_w, beta_b):
    b = feature_A.shape[0]
    fA = feature_A.reshape(b, _C, _P)
    fB = feature_B.reshape(b, _C, _P)
    sA = segmap_A.astype(jnp.int32).reshape(b, 1, _P)
    sB = segmap_B.astype(jnp.int32).reshape(b, 1, _P)
    fcwp = jnp.transpose(fc_w, (1, 0, 2))
    fcbt = jnp.transpose(fc_b)

    vecs = pl.pallas_call(
        _pool_body,
        grid=(b,),
        in_specs=[
            pl.BlockSpec((1, _C, _P), lambda i: (i, 0, 0)),
            pl.BlockSpec((1, _C, _P), lambda i: (i, 0, 0)),
            pl.BlockSpec((1, 1, _P), lambda i: (i, 0, 0)),
            pl.BlockSpec((1, 1, _P), lambda i: (i, 0, 0)),
            pl.BlockSpec((_C, _S, _C), lambda i: (0, 0, 0)),
            pl.BlockSpec((_C, _S), lambda i: (0, 0)),
        ],
        out_specs=pl.BlockSpec((2, 1, _C, _S), lambda i: (0, i, 0, 0)),
        out_shape=jax.ShapeDtypeStruct((2, b, _C, _S), jnp.float32),
    )(fA, fB, sA, sB, fcwp, fcbt)

    w2 = jnp.concatenate([gamma_w, beta_w], axis=0)
    wt = jnp.transpose(w2, (2, 3, 1, 0)).reshape(9, _C, 2 * _C)
    bias2 = jnp.concatenate([gamma_b, beta_b])[None, :]

    out = pl.pallas_call(
        _conv_body,
        grid=(4, b, _NRB),
        in_specs=[
            pl.BlockSpec((1, 1, _P), lambda p, i, r: (i, 0, 0)),
            pl.BlockSpec((1, 1, _P), lambda p, i, r: (i, 0, 0)),
            pl.BlockSpec((1, 1, _C, _S), lambda p, i, r: (_tgt_idx(p), i, 0, 0)),
            pl.BlockSpec((9, _C, 2 * _C), lambda p, i, r: (0, 0, 0)),
            pl.BlockSpec((1, 2 * _C), lambda p, i, r: (0, 0)),
        ],
        out_specs=pl.BlockSpec((2, 1, 1, _C, _RB * _W),
                               lambda p, i, r: (0, p, i, 0, r)),
        out_shape=jax.ShapeDtypeStruct((2, 4, b, _C, _P), jnp.float32),
        scratch_shapes=[pltpu.VMEM((_H + 2, _W + 2, _C), jnp.float32)],
    )(sA, sB, vecs, wt, bias2)

    return out.reshape(8, b, _C, _H, _W)


# speedup vs baseline: 2.4715x; 1.2884x over previous
import jax
import jax.numpy as jnp
from jax import lax
from jax.experimental import pallas as pl
from jax.experimental.pallas import tpu as pltpu

_NCLS = 19
_S = _NCLS + 1
_C = 64
_H = 128
_W = 128
_P = _H * _W
_RB = 32
_NRB = _H // _RB


def _pool_one(x, seg_r, fcwp_ref, fcbt_ref):
    f32 = jnp.float32
    iota_sr = lax.broadcasted_iota(jnp.int32, (_S, _P), 0)
    oh = (seg_r == iota_sr - 1).astype(f32)
    sums = lax.dot_general(x, oh, (((1,), (1,)), ((), ())),
                           preferred_element_type=f32)
    ones_row = jnp.ones((1, _P), f32)
    area = lax.dot_general(ones_row, oh, (((1,), (1,)), ((), ())),
                           preferred_element_type=f32)
    mu = jnp.where(area > 0.0, sums / jnp.maximum(area, 1.0), 0.0)
    glob = jnp.sum(sums, axis=1, keepdims=True) * (1.0 / _P)

    fcwp_flat = fcwp_ref[...].reshape(_C * _S, _C)
    prod = lax.dot_general(fcwp_flat, mu, (((1,), (0,)), ((), ())),
                           preferred_element_type=f32)
    prod3 = prod.reshape(_C, _S, _S)
    sel = (lax.broadcasted_iota(jnp.int32, (_C, _S, _S), 1) ==
           lax.broadcasted_iota(jnp.int32, (_C, _S, _S), 2))
    fc_lin = jnp.sum(jnp.where(sel, prod3, 0.0), axis=2) + fcbt_ref[...]
    fc_out = jnp.maximum(fc_lin, 0.0)

    fc0 = jnp.maximum(
        lax.dot_general(fcwp_ref[:, 0, :], glob, (((1,), (0,)), ((), ())),
                        preferred_element_type=f32) + fcbt_ref[:, 0:1], 0.0)

    present = jnp.sum(mu, axis=0, keepdims=True)
    sidx = lax.broadcasted_iota(jnp.int32, (1, _S), 1)
    use_fc = jnp.logical_and(sidx != 0, present != 0.0)
    return jnp.where(use_fc, fc_out, fc0)


def _pool_body(featA_ref, featB_ref, segA_ref, segB_ref, fcwp_ref, fcbt_ref,
               vec_ref):
    vec_ref[0, 0] = _pool_one(featA_ref[0], segA_ref[0], fcwp_ref, fcbt_ref)
    vec_ref[1, 0] = _pool_one(featB_ref[0], segB_ref[0], fcwp_ref, fcbt_ref)


def _conv_body(segA_ref, segB_ref, vec_ref, wt_ref, bias_ref, out_ref,
               fpad_ref):
    f32 = jnp.float32
    pid0 = pl.program_id(0)
    pid1 = pl.program_id(1)
    rb = pl.program_id(2)

    @pl.when(jnp.logical_and(jnp.logical_and(pid0 == 0, pid1 == 0), rb == 0))
    def _zero():
        fpad_ref[...] = jnp.zeros((_H + 2, _W + 2, _C), f32)

    @pl.when(rb == 0)
    def _build_f():
        seg_c = jnp.where(pid0 % 2 == 0, segA_ref[0], segB_ref[0])
        iota_sr = lax.broadcasted_iota(jnp.int32, (_S, _P), 0)
        oh_t = (seg_c == iota_sr - 1).astype(f32)
        oh = jnp.transpose(oh_t)
        f = lax.dot_general(oh, vec_ref[0, 0], (((1,), (1,)), ((), ())),
                            preferred_element_type=f32)
        fpad_ref[1:_H + 1, 1:_W + 1, :] = f.reshape(_H, _W, _C)

    acc = None
    for t in range(9):
        ky, kx = divmod(t, 3)
        xs = fpad_ref[pl.ds(rb * _RB + ky, _RB), kx:kx + _W, :]
        xs = xs.reshape(_RB * _W, _C)
        m = lax.dot_general(xs, wt_ref[t], (((1,), (0,)), ((), ())),
                            preferred_element_type=f32)
        acc = m if acc is None else acc + m
    res = jnp.transpose(jnp.maximum(acc + bias_ref[...], 0.0))
    v3 = res.reshape(2 * _C, _RB, _W)
    out_ref[0, 0, 0] = v3[:_C]
    out_ref[1, 0, 0] = v3[_C:]


def _src_idx(p):
    return p % 2


def _tgt_idx(p):
    return (p % 2 + p // 2) % 2


def kernel(feature_A, feature_B, segmap_A, segmap_B, fc_w, fc_b,
           gamma_w, gamma_b, beta_w, beta_b):
    b = feature_A.shape[0]
    fA = feature_A.reshape(b, _C, _P)
    fB = feature_B.reshape(b, _C, _P)
    sA = segmap_A.astype(jnp.int32).reshape(b, 1, _P)
    sB = segmap_B.astype(jnp.int32).reshape(b, 1, _P)
    fcwp = jnp.transpose(fc_w, (1, 0, 2))
    fcbt = jnp.transpose(fc_b)

    vecs = pl.pallas_call(
        _pool_body,
        grid=(b,),
        in_specs=[
            pl.BlockSpec((1, _C, _P), lambda i: (i, 0, 0)),
            pl.BlockSpec((1, _C, _P), lambda i: (i, 0, 0)),
            pl.BlockSpec((1, 1, _P), lambda i: (i, 0, 0)),
            pl.BlockSpec((1, 1, _P), lambda i: (i, 0, 0)),
            pl.BlockSpec((_C, _S, _C), lambda i: (0, 0, 0)),
            pl.BlockSpec((_C, _S), lambda i: (0, 0)),
        ],
        out_specs=pl.BlockSpec((2, 1, _C, _S), lambda i: (0, i, 0, 0)),
        out_shape=jax.ShapeDtypeStruct((2, b, _C, _S), jnp.float32),
    )(fA, fB, sA, sB, fcwp, fcbt)

    w2 = jnp.concatenate([gamma_w, beta_w], axis=0)
    wt = jnp.transpose(w2, (2, 3, 1, 0)).reshape(9, _C, 2 * _C)
    bias2 = jnp.concatenate([gamma_b, beta_b])[None, :]

    out = pl.pallas_call(
        _conv_body,
        grid=(4, b, _NRB),
        in_specs=[
            pl.BlockSpec((1, 1, _P), lambda p, i, r: (i, 0, 0)),
            pl.BlockSpec((1, 1, _P), lambda p, i, r: (i, 0, 0)),
            pl.BlockSpec((1, 1, _C, _S), lambda p, i, r: (_tgt_idx(p), i, 0, 0)),
            pl.BlockSpec((9, _C, 2 * _C), lambda p, i, r: (0, 0, 0)),
            pl.BlockSpec((1, 2 * _C), lambda p, i, r: (0, 0)),
        ],
        out_specs=pl.BlockSpec((2, 1, 1, _C, _RB, _W),
                               lambda p, i, r: (0, p, i, 0, r, 0)),
        out_shape=jax.ShapeDtypeStruct((2, 4, b, _C, _H, _W), jnp.float32),
        scratch_shapes=[pltpu.VMEM((_H + 2, _W + 2, _C), jnp.float32)],
    )(sA, sB, vecs, wt, bias2)

    return out.reshape(8, b, _C, _H, _W)
